# baseline (speedup 1.0000x reference)
import jax
import jax.numpy as jnp
from jax import lax
from jax.experimental import pallas as pl
from jax.experimental.pallas import tpu as pltpu

NUM_QUANTIZERS = 8
CODEBOOK_SIZE = 1024
DIM = 64
T_BLK = 512


def _split_kernel(cb_ref, hi2_ref, pack_ref):
    cb = cb_ref[...]
    hi = cb.astype(jnp.bfloat16)
    rem = cb - hi.astype(jnp.float32)
    mid = rem.astype(jnp.bfloat16)
    lo = (rem - mid.astype(jnp.float32)).astype(jnp.bfloat16)
    hi2_ref[...] = (-2.0 * hi.astype(jnp.float32)).astype(jnp.bfloat16)
    pack_ref[...] = jnp.concatenate([hi, mid, lo], axis=2)


def _rvq_kernel(x_ref, cbh2_ref, cbp_ref, cbn_ref, out_ref):
    r = x_ref[0]
    acc = jnp.zeros_like(r)
    t_blk = r.shape[1]
    iota = lax.broadcasted_iota(jnp.int32, (t_blk, CODEBOOK_SIZE), 1)
    for q in range(NUM_QUANTIZERS):
        cb_hi2 = cbh2_ref[q]
        cb_pack = cbp_ref[q]
        cbn = cbn_ref[q]
        rn = jnp.sum(r * r, axis=0)
        s2 = lax.dot_general(
            r.astype(jnp.bfloat16), cb_hi2, (((0,), (1,)), ((), ())),
            preferred_element_type=jnp.float32,
        )
        dist = (rn[:, None] + s2) + cbn[None, :]
        mval = jnp.min(dist, axis=1)
        ind = jnp.min(
            jnp.where(dist == mval[:, None], iota, CODEBOOK_SIZE), axis=1
        )
        onehot = (iota == ind[:, None]).astype(jnp.bfloat16)
        g = lax.dot_general(
            cb_pack, onehot, (((0,), (1,)), ((), ())),
            preferred_element_type=jnp.float32,
        )
        qv = (g[0:DIM] + g[DIM:2 * DIM]) + g[2 * DIM:3 * DIM]
        qv_st = r + (qv - r)
        r = r - qv_st
        acc = acc + qv_st
    out_ref[0] = acc


@jax.jit
def kernel(x, codebooks):
    B, D, T = x.shape
    cb_hi2, cb_pack = pl.pallas_call(
        _split_kernel,
        out_shape=[
            jax.ShapeDtypeStruct(
                (NUM_QUANTIZERS, CODEBOOK_SIZE, DIM), jnp.bfloat16
            ),
            jax.ShapeDtypeStruct(
                (NUM_QUANTIZERS, CODEBOOK_SIZE, 3 * DIM), jnp.bfloat16
            ),
        ],
    )(codebooks)
    cbn = jnp.sum(codebooks * codebooks, axis=-1)
    grid = (B, T // T_BLK)
    return pl.pallas_call(
        _rvq_kernel,
        grid=grid,
        in_specs=[
            pl.BlockSpec((1, D, T_BLK), lambda b, t: (b, 0, t)),
            pl.BlockSpec(
                (NUM_QUANTIZERS, CODEBOOK_SIZE, DIM), lambda b, t: (0, 0, 0)
            ),
            pl.BlockSpec(
                (NUM_QUANTIZERS, CODEBOOK_SIZE, 3 * DIM),
                lambda b, t: (0, 0, 0),
            ),
            pl.BlockSpec((NUM_QUANTIZERS, CODEBOOK_SIZE), lambda b, t: (0, 0)),
        ],
        out_specs=pl.BlockSpec((1, D, T_BLK), lambda b, t: (b, 0, t)),
        out_shape=jax.ShapeDtypeStruct((B, D, T), jnp.float32),
        compiler_params=pltpu.CompilerParams(
            dimension_semantics=("parallel", "parallel")
        ),
    )(x, cb_hi2, cb_pack, cbn)


# speedup vs baseline: 1.6296x; 1.6296x over previous
import jax
import jax.numpy as jnp
from jax import lax
from jax.experimental import pallas as pl
from jax.experimental.pallas import tpu as pltpu

NUM_QUANTIZERS = 8
CODEBOOK_SIZE = 1024
DIM = 64
T_BLK = 2048


def _split_kernel(cb_ref, hi2_ref, pack_ref):
    cb = cb_ref[...]
    hi = cb.astype(jnp.bfloat16)
    rem = cb - hi.astype(jnp.float32)
    mid = rem.astype(jnp.bfloat16)
    lo = (rem - mid.astype(jnp.float32)).astype(jnp.bfloat16)
    hi2_ref[...] = (-2.0 * hi.astype(jnp.float32)).astype(jnp.bfloat16)
    pack_ref[...] = jnp.concatenate([hi, mid, lo], axis=2)


def _rvq_kernel(x_ref, cbh2_ref, cbp_ref, cbn_ref, out_ref):
    r = x_ref[0]
    acc = jnp.zeros_like(r)
    t_blk = r.shape[1]
    iota = lax.broadcasted_iota(jnp.int32, (t_blk, CODEBOOK_SIZE), 1)
    for q in range(NUM_QUANTIZERS):
        cb_hi2 = cbh2_ref[q]
        cb_pack = cbp_ref[q]
        cbn = cbn_ref[q]
        rn = jnp.sum(r * r, axis=0)
        s2 = lax.dot_general(
            r.astype(jnp.bfloat16), cb_hi2, (((0,), (1,)), ((), ())),
            preferred_element_type=jnp.float32,
        )
        dist = (rn[:, None] + s2) + cbn[None, :]
        mval = jnp.min(dist, axis=1)
        ind = jnp.min(
            jnp.where(dist == mval[:, None], iota, CODEBOOK_SIZE), axis=1
        )
        onehot = (iota == ind[:, None]).astype(jnp.bfloat16)
        g = lax.dot_general(
            cb_pack, onehot, (((0,), (1,)), ((), ())),
            preferred_element_type=jnp.float32,
        )
        qv = (g[0:DIM] + g[DIM:2 * DIM]) + g[2 * DIM:3 * DIM]
        qv_st = r + (qv - r)
        r = r - qv_st
        acc = acc + qv_st
    out_ref[0] = acc


@jax.jit
def kernel(x, codebooks):
    B, D, T = x.shape
    cb_hi2, cb_pack = pl.pallas_call(
        _split_kernel,
        out_shape=[
            jax.ShapeDtypeStruct(
                (NUM_QUANTIZERS, CODEBOOK_SIZE, DIM), jnp.bfloat16
            ),
            jax.ShapeDtypeStruct(
                (NUM_QUANTIZERS, CODEBOOK_SIZE, 3 * DIM), jnp.bfloat16
            ),
        ],
    )(codebooks)
    cbn = jnp.sum(codebooks * codebooks, axis=-1)
    grid = (B, T // T_BLK)
    return pl.pallas_call(
        _rvq_kernel,
        grid=grid,
        in_specs=[
            pl.BlockSpec((1, D, T_BLK), lambda b, t: (b, 0, t)),
            pl.BlockSpec(
                (NUM_QUANTIZERS, CODEBOOK_SIZE, DIM), lambda b, t: (0, 0, 0)
            ),
            pl.BlockSpec(
                (NUM_QUANTIZERS, CODEBOOK_SIZE, 3 * DIM),
                lambda b, t: (0, 0, 0),
            ),
            pl.BlockSpec((NUM_QUANTIZERS, CODEBOOK_SIZE), lambda b, t: (0, 0)),
        ],
        out_specs=pl.BlockSpec((1, D, T_BLK), lambda b, t: (b, 0, t)),
        out_shape=jax.ShapeDtypeStruct((B, D, T), jnp.float32),
        compiler_params=pltpu.CompilerParams(
            dimension_semantics=("parallel", "parallel")
        ),
    )(x, cb_hi2, cb_pack, cbn)
